# Initial kernel scaffold; baseline (speedup 1.0000x reference)
import jax
import jax.numpy as jnp
from jax.experimental import pallas as pl


def kernel(x, edge_index, batch, protein, drug_emb, W1, a_src1, a_dst1, b1, W2, a_src2, a_dst2, b2, W3, a_src3, a_dst3, b3, g3, be3, prot_emb, pw1, pb1, pg1, pbe1, pw2, pb2, pg2, pbe2, pw3, pb3, pg3, pbe3, fW1, fb1, fg1, fbe1, fW2, fb2, fg2, fbe2, oW, ob):
    raise NotImplementedError("write your pallas kernel here")


# speedup vs baseline: 3.2201x; 3.2201x over previous
import functools
import jax
import jax.numpy as jnp
from jax.experimental import pallas as pl

N_NODES = 50000
NODE_BLK = 5000
PB = 10
PLEN = 1000


def _gat_transform_body(x_ref, w_ref, as_ref, ad_ref, b_ref, h_ref, als_ref,
                        ald_ref, *, prologue):
    x = x_ref[...]
    if prologue:
        t = x + b_ref[...]
        x = jnp.where(t > 0, t, jnp.exp(t) - 1.0)
    h = jnp.dot(x, w_ref[...], preferred_element_type=jnp.float32)
    h_ref[...] = h
    als_ref[...] = jnp.dot(h, as_ref[...], preferred_element_type=jnp.float32)
    ald_ref[...] = jnp.dot(h, ad_ref[...], preferred_element_type=jnp.float32)


def _gat_transform(x, W, As, Ad, b_prev, prologue):
    n, cin = x.shape
    hc = W.shape[1]
    grid = n // NODE_BLK
    body = functools.partial(_gat_transform_body, prologue=prologue)
    return pl.pallas_call(
        body,
        grid=(grid,),
        in_specs=[
            pl.BlockSpec((NODE_BLK, cin), lambda i: (i, 0)),
            pl.BlockSpec((cin, hc), lambda i: (0, 0)),
            pl.BlockSpec((hc, 8), lambda i: (0, 0)),
            pl.BlockSpec((hc, 8), lambda i: (0, 0)),
            pl.BlockSpec((1, cin), lambda i: (0, 0)),
        ],
        out_specs=[
            pl.BlockSpec((NODE_BLK, hc), lambda i: (i, 0)),
            pl.BlockSpec((NODE_BLK, 8), lambda i: (i, 0)),
            pl.BlockSpec((NODE_BLK, 8), lambda i: (i, 0)),
        ],
        out_shape=[
            jax.ShapeDtypeStruct((n, hc), jnp.float32),
            jax.ShapeDtypeStruct((n, 8), jnp.float32),
            jax.ShapeDtypeStruct((n, 8), jnp.float32),
        ],
    )(x, W, As, Ad, b_prev)


def _conv_body(x_ref, w_ref, b_ref, sc_ref, sh_ref, y_ref, s_ref, q_ref, *,
               lin_valid, prologue, cin, cout):
    x = x_ref[...]
    if prologue:
        x = x * sc_ref[...] + sh_ref[...]
        x = jnp.maximum(x, 0.0)
        pos = jax.lax.broadcasted_iota(jnp.int32, (1, PLEN, 1), 1)
        x = jnp.where(pos < lin_valid, x, 0.0)
    x2 = x.reshape(PB * PLEN, cin)
    w = w_ref[...]
    y0 = jnp.dot(x2, w[0:cin, :], preferred_element_type=jnp.float32)
    y1 = jnp.dot(x2, w[cin:2 * cin, :], preferred_element_type=jnp.float32)
    y2 = jnp.dot(x2, w[2 * cin:3 * cin, :], preferred_element_type=jnp.float32)
    y0 = y0.reshape(PB, PLEN, cout)
    y1 = y1.reshape(PB, PLEN, cout)
    y2 = y2.reshape(PB, PLEN, cout)
    lout = lin_valid - 2
    y = (y0[:, 0:PLEN - 2, :] + y1[:, 1:PLEN - 1, :] + y2[:, 2:PLEN, :]
         + b_ref[...])
    pos = jax.lax.broadcasted_iota(jnp.int32, (1, PLEN - 2, 1), 1)
    y = jnp.where(pos < lout, y, 0.0)
    y_ref[:, 0:PLEN - 2, :] = y
    y_ref[:, PLEN - 2:PLEN, :] = jnp.zeros((PB, 2, cout), jnp.float32)
    s_ref[...] = jnp.sum(y, axis=(0, 1)).reshape(1, 1, cout)
    q_ref[...] = jnp.sum(y * y, axis=(0, 1)).reshape(1, 1, cout)


def _conv_bn(x, w, b, scale, shift, lin_valid, prologue):
    bsz = x.shape[0]
    cin = x.shape[2]
    cout = w.shape[1]
    grid = bsz // PB
    body = functools.partial(_conv_body, lin_valid=lin_valid,
                             prologue=prologue, cin=cin, cout=cout)
    return pl.pallas_call(
        body,
        grid=(grid,),
        in_specs=[
            pl.BlockSpec((PB, PLEN, cin), lambda i: (i, 0, 0)),
            pl.BlockSpec((3 * cin, cout), lambda i: (0, 0)),
            pl.BlockSpec((1, 1, cout), lambda i: (0, 0, 0)),
            pl.BlockSpec((1, 1, cin), lambda i: (0, 0, 0)),
            pl.BlockSpec((1, 1, cin), lambda i: (0, 0, 0)),
        ],
        out_specs=[
            pl.BlockSpec((PB, PLEN, cout), lambda i: (i, 0, 0)),
            pl.BlockSpec((1, 1, cout), lambda i: (i, 0, 0)),
            pl.BlockSpec((1, 1, cout), lambda i: (i, 0, 0)),
        ],
        out_shape=[
            jax.ShapeDtypeStruct((bsz, PLEN, cout), jnp.float32),
            jax.ShapeDtypeStruct((grid, 1, cout), jnp.float32),
            jax.ShapeDtypeStruct((grid, 1, cout), jnp.float32),
        ],
    )(x, w, b, scale, shift)


def _bn_stats(s, q, count, g, b):
    mu = jnp.sum(s, axis=0) / count
    var = jnp.sum(q, axis=0) / count - mu * mu
    scale = g / jnp.sqrt(var + 1e-5)
    shift = b - mu * scale
    return scale, shift


def _bnmax_body(x_ref, sc_ref, sh_ref, o_ref, *, lvalid, c):
    x = x_ref[...]
    x = jnp.maximum(x * sc_ref[...] + sh_ref[...], 0.0)
    pos = jax.lax.broadcasted_iota(jnp.int32, (1, PLEN, 1), 1)
    x = jnp.where(pos < lvalid, x, -1e30)
    o_ref[...] = jnp.max(x, axis=1).reshape(1, -1, c)


def _bn_relu_max(x, scale, shift, lvalid):
    bsz, _, c = x.shape
    grid = bsz // PB
    body = functools.partial(_bnmax_body, lvalid=lvalid, c=c)
    return pl.pallas_call(
        body,
        grid=(grid,),
        in_specs=[
            pl.BlockSpec((PB, PLEN, c), lambda i: (i, 0, 0)),
            pl.BlockSpec((1, 1, c), lambda i: (0, 0, 0)),
            pl.BlockSpec((1, 1, c), lambda i: (0, 0, 0)),
        ],
        out_specs=pl.BlockSpec((1, PB, c), lambda i: (i, 0, 0)),
        out_shape=jax.ShapeDtypeStruct((grid, PB, c), jnp.float32),
    )(x, scale, shift).reshape(bsz, c)


def _bn_cols(x, g, b):
    mu = jnp.mean(x, axis=0, keepdims=True)
    var = jnp.mean((x - mu) * (x - mu), axis=0, keepdims=True)
    return g * (x - mu) / jnp.sqrt(var + 1e-5) + b


def _fusion_body(summed_ref, ic_ref, g3_ref, be3_ref, pv_ref, fw1a_ref,
                 fw1b_ref, fb1_ref, fg1_ref, fbe1_ref, fw2_ref, fb2_ref,
                 fg2_ref, fbe2_ref, ow_ref, o_ref):
    dv = summed_ref[...] * ic_ref[...]
    dv = _bn_cols(dv, g3_ref[...], be3_ref[...])
    z = (jnp.dot(dv, fw1a_ref[...], preferred_element_type=jnp.float32)
         + jnp.dot(pv_ref[...], fw1b_ref[...],
                   preferred_element_type=jnp.float32) + fb1_ref[...])
    z = jnp.maximum(_bn_cols(z, fg1_ref[...], fbe1_ref[...]), 0.0)
    z = jnp.dot(z, fw2_ref[...], preferred_element_type=jnp.float32) + fb2_ref[...]
    z = jnp.maximum(_bn_cols(z, fg2_ref[...], fbe2_ref[...]), 0.0)
    o_ref[...] = jnp.dot(z, ow_ref[...], preferred_element_type=jnp.float32)


def _fusion_head(summed, inv_counts, g3, be3, prot_vec, fW1a, fW1b, fb1, fg1,
                 fbe1, fW2, fb2, fg2, fbe2, oWp):
    bsz = summed.shape[0]
    full = lambda s: pl.BlockSpec(s, lambda: tuple(0 for _ in s))
    return pl.pallas_call(
        _fusion_body,
        in_specs=[
            full((bsz, 128)), full((bsz, 128)), full((1, 128)),
            full((1, 128)), full((bsz, 128)), full((128, 128)),
            full((128, 128)), full((1, 128)), full((1, 128)), full((1, 128)),
            full((128, 64)), full((1, 64)), full((1, 64)), full((1, 64)),
            full((64, 8)),
        ],
        out_specs=full((bsz, 8)),
        out_shape=jax.ShapeDtypeStruct((bsz, 8), jnp.float32),
    )(summed, inv_counts, g3, be3, prot_vec, fW1a, fW1b, fb1, fg1, fbe1, fW2,
      fb2, fg2, fbe2, oWp)


def _att_mat(a, H, C):
    m = jnp.zeros((H * C, 8), jnp.float32)
    for j in range(H):
        m = m.at[j * C:(j + 1) * C, j].set(a[j])
    return m


def _edge_softmax_agg(h, als, ald, src, dst, H, C, n):
    e = als[src] + ald[dst]
    e = jnp.where(e > 0, e, 0.2 * e)
    m = jax.ops.segment_max(e, dst, num_segments=n)
    m = jnp.where(jnp.isfinite(m), m, 0.0)
    ex = jnp.exp(e - m[dst])
    s = jax.ops.segment_sum(ex, dst, num_segments=n)
    alpha = ex / (s[dst] + 1e-16)
    if C > 1:
        alpha = jnp.repeat(alpha, C, axis=1)
    return jax.ops.segment_sum(h[src] * alpha, dst, num_segments=n)


def kernel(x, edge_index, batch, protein, drug_emb, W1, a_src1, a_dst1, b1,
           W2, a_src2, a_dst2, b2, W3, a_src3, a_dst3, b3, g3, be3, prot_emb,
           pw1, pb1, pg1, pbe1, pw2, pb2, pg2, pbe2, pw3, pb3, pg3, pbe3,
           fW1, fb1, fg1, fbe1, fW2, fb2, fg2, fbe2, oW, ob):
    n = x.shape[0]
    loop = jnp.arange(n, dtype=edge_index.dtype)
    src = jnp.concatenate([edge_index[0], loop])
    dst = jnp.concatenate([edge_index[1], loop])

    h0 = drug_emb[x]
    row = lambda v: v.reshape(1, -1)

    h, als, ald = _gat_transform(h0, W1, _att_mat(a_src1, 2, 32),
                                 _att_mat(a_dst1, 2, 32),
                                 jnp.zeros((1, 32), jnp.float32), False)
    s1 = _edge_softmax_agg(h, als[:, :2], ald[:, :2], src, dst, 2, 32, n)

    h, als, ald = _gat_transform(s1, W2, _att_mat(a_src2, 2, 64),
                                 _att_mat(a_dst2, 2, 64), row(b1), True)
    s2 = _edge_softmax_agg(h, als[:, :2], ald[:, :2], src, dst, 2, 64, n)

    h, als, ald = _gat_transform(s2, W3, _att_mat(a_src3, 1, 128),
                                 _att_mat(a_dst3, 1, 128), row(b2), True)
    s3 = _edge_softmax_agg(h, als[:, :1], ald[:, :1], src, dst, 1, 128, n)

    t3 = s3 + b3[None, :]
    h3 = jnp.where(t3 > 0, t3, jnp.exp(t3) - 1.0)

    bsz = protein.shape[0]
    counts = jax.ops.segment_sum(jnp.ones((n,), jnp.float32), batch,
                                 num_segments=bsz)
    summed = jax.ops.segment_sum(h3, batch, num_segments=bsz)
    inv_counts = (1.0 / jnp.clip(counts, 1.0))[:, None] * jnp.ones(
        (1, 128), jnp.float32)

    p0 = prot_emb[protein]
    w1m = pw1.transpose(2, 1, 0).reshape(3 * 32, 64)
    w2m = pw2.transpose(2, 1, 0).reshape(3 * 64, 128)
    w3m = pw3.transpose(2, 1, 0).reshape(3 * 128, 128)
    zc32 = jnp.zeros((1, 1, 32), jnp.float32)
    y1, s_1, q_1 = _conv_bn(p0, w1m, pb1.reshape(1, 1, 64), zc32, zc32,
                            PLEN, False)
    sc1, sh1 = _bn_stats(s_1, q_1, bsz * 998.0, pg1, pbe1)
    y2, s_2, q_2 = _conv_bn(y1, w2m, pb2.reshape(1, 1, 128),
                            sc1.reshape(1, 1, 64), sh1.reshape(1, 1, 64),
                            998, True)
    sc2, sh2 = _bn_stats(s_2, q_2, bsz * 996.0, pg2, pbe2)
    y3, s_3, q_3 = _conv_bn(y2, w3m, pb3.reshape(1, 1, 128),
                            sc2.reshape(1, 1, 128), sh2.reshape(1, 1, 128),
                            996, True)
    sc3, sh3 = _bn_stats(s_3, q_3, bsz * 994.0, pg3, pbe3)
    prot_vec = _bn_relu_max(y3, sc3.reshape(1, 1, 128),
                            sh3.reshape(1, 1, 128), 994)

    oWp = jnp.concatenate([oW, jnp.zeros((64, 7), jnp.float32)], axis=1)
    out = _fusion_head(summed, inv_counts, row(g3), row(be3), prot_vec,
                       fW1[:128], fW1[128:], row(fb1), row(fg1), row(fbe1),
                       fW2, row(fb2), row(fg2), row(fbe2), oWp)
    return out[:, :1] + ob[None, :]


# speedup vs baseline: 5.3482x; 1.6609x over previous
import functools
import jax
import jax.numpy as jnp
from jax.experimental import pallas as pl

N_NODES = 50000
NODE_BLK = 5000
PB = 10
PLEN = 1000


def _gat_transform_body(x_ref, w_ref, as_ref, ad_ref, b_ref, h_ref, als_ref,
                        ald_ref, *, prologue):
    x = x_ref[...]
    if prologue:
        t = x + b_ref[...]
        x = jnp.where(t > 0, t, jnp.exp(t) - 1.0)
    h = jnp.dot(x, w_ref[...], preferred_element_type=jnp.float32)
    h_ref[...] = h
    als_ref[...] = jnp.dot(h, as_ref[...], preferred_element_type=jnp.float32)
    ald_ref[...] = jnp.dot(h, ad_ref[...], preferred_element_type=jnp.float32)


def _gat_transform(x, W, As, Ad, b_prev, prologue):
    n, cin = x.shape
    hc = W.shape[1]
    grid = n // NODE_BLK
    body = functools.partial(_gat_transform_body, prologue=prologue)
    return pl.pallas_call(
        body,
        grid=(grid,),
        in_specs=[
            pl.BlockSpec((NODE_BLK, cin), lambda i: (i, 0)),
            pl.BlockSpec((cin, hc), lambda i: (0, 0)),
            pl.BlockSpec((hc, 8), lambda i: (0, 0)),
            pl.BlockSpec((hc, 8), lambda i: (0, 0)),
            pl.BlockSpec((1, cin), lambda i: (0, 0)),
        ],
        out_specs=[
            pl.BlockSpec((NODE_BLK, hc), lambda i: (i, 0)),
            pl.BlockSpec((NODE_BLK, 8), lambda i: (i, 0)),
            pl.BlockSpec((NODE_BLK, 8), lambda i: (i, 0)),
        ],
        out_shape=[
            jax.ShapeDtypeStruct((n, hc), jnp.float32),
            jax.ShapeDtypeStruct((n, 8), jnp.float32),
            jax.ShapeDtypeStruct((n, 8), jnp.float32),
        ],
    )(x, W, As, Ad, b_prev)


def _conv_body(x_ref, w_ref, b_ref, sc_ref, sh_ref, y_ref, s_ref, q_ref, *,
               lin_valid, prologue, cin, cout):
    x = x_ref[...]
    if prologue:
        x = x * sc_ref[...] + sh_ref[...]
        x = jnp.maximum(x, 0.0)
        pos = jax.lax.broadcasted_iota(jnp.int32, (1, PLEN, 1), 1)
        x = jnp.where(pos < lin_valid, x, 0.0)
    x2 = x.reshape(PB * PLEN, cin)
    w = w_ref[...]
    y0 = jnp.dot(x2, w[0:cin, :], preferred_element_type=jnp.float32)
    y1 = jnp.dot(x2, w[cin:2 * cin, :], preferred_element_type=jnp.float32)
    y2 = jnp.dot(x2, w[2 * cin:3 * cin, :], preferred_element_type=jnp.float32)
    y0 = y0.reshape(PB, PLEN, cout)
    y1 = y1.reshape(PB, PLEN, cout)
    y2 = y2.reshape(PB, PLEN, cout)
    lout = lin_valid - 2
    y = (y0[:, 0:PLEN - 2, :] + y1[:, 1:PLEN - 1, :] + y2[:, 2:PLEN, :]
         + b_ref[...])
    pos = jax.lax.broadcasted_iota(jnp.int32, (1, PLEN - 2, 1), 1)
    y = jnp.where(pos < lout, y, 0.0)
    y_ref[:, 0:PLEN - 2, :] = y
    y_ref[:, PLEN - 2:PLEN, :] = jnp.zeros((PB, 2, cout), jnp.float32)
    s_ref[...] = jnp.sum(y, axis=(0, 1)).reshape(1, 1, cout)
    q_ref[...] = jnp.sum(y * y, axis=(0, 1)).reshape(1, 1, cout)


def _conv_bn(x, w, b, scale, shift, lin_valid, prologue):
    bsz = x.shape[0]
    cin = x.shape[2]
    cout = w.shape[1]
    grid = bsz // PB
    body = functools.partial(_conv_body, lin_valid=lin_valid,
                             prologue=prologue, cin=cin, cout=cout)
    return pl.pallas_call(
        body,
        grid=(grid,),
        in_specs=[
            pl.BlockSpec((PB, PLEN, cin), lambda i: (i, 0, 0)),
            pl.BlockSpec((3 * cin, cout), lambda i: (0, 0)),
            pl.BlockSpec((1, 1, cout), lambda i: (0, 0, 0)),
            pl.BlockSpec((1, 1, cin), lambda i: (0, 0, 0)),
            pl.BlockSpec((1, 1, cin), lambda i: (0, 0, 0)),
        ],
        out_specs=[
            pl.BlockSpec((PB, PLEN, cout), lambda i: (i, 0, 0)),
            pl.BlockSpec((1, 1, cout), lambda i: (i, 0, 0)),
            pl.BlockSpec((1, 1, cout), lambda i: (i, 0, 0)),
        ],
        out_shape=[
            jax.ShapeDtypeStruct((bsz, PLEN, cout), jnp.float32),
            jax.ShapeDtypeStruct((grid, 1, cout), jnp.float32),
            jax.ShapeDtypeStruct((grid, 1, cout), jnp.float32),
        ],
    )(x, w, b, scale, shift)


def _bn_stats(s, q, count, g, b):
    mu = jnp.sum(s, axis=0) / count
    var = jnp.sum(q, axis=0) / count - mu * mu
    scale = g / jnp.sqrt(var + 1e-5)
    shift = b - mu * scale
    return scale, shift


def _bnmax_body(x_ref, sc_ref, sh_ref, o_ref, *, lvalid, c):
    x = x_ref[...]
    x = jnp.maximum(x * sc_ref[...] + sh_ref[...], 0.0)
    pos = jax.lax.broadcasted_iota(jnp.int32, (1, PLEN, 1), 1)
    x = jnp.where(pos < lvalid, x, -1e30)
    o_ref[...] = jnp.max(x, axis=1).reshape(1, -1, c)


def _bn_relu_max(x, scale, shift, lvalid):
    bsz, _, c = x.shape
    grid = bsz // PB
    body = functools.partial(_bnmax_body, lvalid=lvalid, c=c)
    return pl.pallas_call(
        body,
        grid=(grid,),
        in_specs=[
            pl.BlockSpec((PB, PLEN, c), lambda i: (i, 0, 0)),
            pl.BlockSpec((1, 1, c), lambda i: (0, 0, 0)),
            pl.BlockSpec((1, 1, c), lambda i: (0, 0, 0)),
        ],
        out_specs=pl.BlockSpec((1, PB, c), lambda i: (i, 0, 0)),
        out_shape=jax.ShapeDtypeStruct((grid, PB, c), jnp.float32),
    )(x, scale, shift).reshape(bsz, c)


def _bn_cols(x, g, b):
    mu = jnp.mean(x, axis=0, keepdims=True)
    var = jnp.mean((x - mu) * (x - mu), axis=0, keepdims=True)
    return g * (x - mu) / jnp.sqrt(var + 1e-5) + b


def _fusion_body(summed_ref, ic_ref, g3_ref, be3_ref, pv_ref, fw1a_ref,
                 fw1b_ref, fb1_ref, fg1_ref, fbe1_ref, fw2_ref, fb2_ref,
                 fg2_ref, fbe2_ref, ow_ref, o_ref):
    dv = summed_ref[...] * ic_ref[...]
    dv = _bn_cols(dv, g3_ref[...], be3_ref[...])
    z = (jnp.dot(dv, fw1a_ref[...], preferred_element_type=jnp.float32)
         + jnp.dot(pv_ref[...], fw1b_ref[...],
                   preferred_element_type=jnp.float32) + fb1_ref[...])
    z = jnp.maximum(_bn_cols(z, fg1_ref[...], fbe1_ref[...]), 0.0)
    z = jnp.dot(z, fw2_ref[...], preferred_element_type=jnp.float32) + fb2_ref[...]
    z = jnp.maximum(_bn_cols(z, fg2_ref[...], fbe2_ref[...]), 0.0)
    o_ref[...] = jnp.dot(z, ow_ref[...], preferred_element_type=jnp.float32)


def _fusion_head(summed, inv_counts, g3, be3, prot_vec, fW1a, fW1b, fb1, fg1,
                 fbe1, fW2, fb2, fg2, fbe2, oWp):
    bsz = summed.shape[0]
    full = lambda s: pl.BlockSpec(s, lambda: tuple(0 for _ in s))
    return pl.pallas_call(
        _fusion_body,
        in_specs=[
            full((bsz, 128)), full((bsz, 128)), full((1, 128)),
            full((1, 128)), full((bsz, 128)), full((128, 128)),
            full((128, 128)), full((1, 128)), full((1, 128)), full((1, 128)),
            full((128, 64)), full((1, 64)), full((1, 64)), full((1, 64)),
            full((64, 8)),
        ],
        out_specs=full((bsz, 8)),
        out_shape=jax.ShapeDtypeStruct((bsz, 8), jnp.float32),
    )(summed, inv_counts, g3, be3, prot_vec, fW1a, fW1b, fb1, fg1, fbe1, fW2,
      fb2, fg2, fbe2, oWp)


def _att_mat(a, H, C):
    m = jnp.zeros((H * C, 8), jnp.float32)
    for j in range(H):
        m = m.at[j * C:(j + 1) * C, j].set(a[j])
    return m


def _edge_softmax_agg(h, als, ald, src, dst, H, C, n):
    e = als[src] + ald[dst]
    e = jnp.where(e > 0, e, 0.2 * e)
    m = jnp.max(e)
    ex = jnp.exp(e - m)
    exr = jnp.repeat(ex, C, axis=1) if C > 1 else ex
    comb = jnp.concatenate([h[src] * exr, ex], axis=1)
    seg = jax.ops.segment_sum(comb, dst, num_segments=n)
    s = seg[:, H * C:]
    sr = jnp.repeat(s, C, axis=1) if C > 1 else s
    return seg[:, :H * C] / (sr + 1e-16)


def kernel(x, edge_index, batch, protein, drug_emb, W1, a_src1, a_dst1, b1,
           W2, a_src2, a_dst2, b2, W3, a_src3, a_dst3, b3, g3, be3, prot_emb,
           pw1, pb1, pg1, pbe1, pw2, pb2, pg2, pbe2, pw3, pb3, pg3, pbe3,
           fW1, fb1, fg1, fbe1, fW2, fb2, fg2, fbe2, oW, ob):
    n = x.shape[0]
    loop = jnp.arange(n, dtype=edge_index.dtype)
    src = jnp.concatenate([edge_index[0], loop])
    dst = jnp.concatenate([edge_index[1], loop])

    h0 = drug_emb[x]
    row = lambda v: v.reshape(1, -1)

    h, als, ald = _gat_transform(h0, W1, _att_mat(a_src1, 2, 32),
                                 _att_mat(a_dst1, 2, 32),
                                 jnp.zeros((1, 32), jnp.float32), False)
    s1 = _edge_softmax_agg(h, als[:, :2], ald[:, :2], src, dst, 2, 32, n)

    h, als, ald = _gat_transform(s1, W2, _att_mat(a_src2, 2, 64),
                                 _att_mat(a_dst2, 2, 64), row(b1), True)
    s2 = _edge_softmax_agg(h, als[:, :2], ald[:, :2], src, dst, 2, 64, n)

    h, als, ald = _gat_transform(s2, W3, _att_mat(a_src3, 1, 128),
                                 _att_mat(a_dst3, 1, 128), row(b2), True)
    s3 = _edge_softmax_agg(h, als[:, :1], ald[:, :1], src, dst, 1, 128, n)

    t3 = s3 + b3[None, :]
    h3 = jnp.where(t3 > 0, t3, jnp.exp(t3) - 1.0)

    bsz = protein.shape[0]
    segb = jax.ops.segment_sum(
        jnp.concatenate([h3, jnp.ones((n, 1), jnp.float32)], axis=1), batch,
        num_segments=bsz)
    summed = segb[:, :128]
    counts = segb[:, 128]
    inv_counts = (1.0 / jnp.clip(counts, 1.0))[:, None] * jnp.ones(
        (1, 128), jnp.float32)

    p0 = prot_emb[protein]
    w1m = pw1.transpose(2, 1, 0).reshape(3 * 32, 64)
    w2m = pw2.transpose(2, 1, 0).reshape(3 * 64, 128)
    w3m = pw3.transpose(2, 1, 0).reshape(3 * 128, 128)
    zc32 = jnp.zeros((1, 1, 32), jnp.float32)
    y1, s_1, q_1 = _conv_bn(p0, w1m, pb1.reshape(1, 1, 64), zc32, zc32,
                            PLEN, False)
    sc1, sh1 = _bn_stats(s_1, q_1, bsz * 998.0, pg1, pbe1)
    y2, s_2, q_2 = _conv_bn(y1, w2m, pb2.reshape(1, 1, 128),
                            sc1.reshape(1, 1, 64), sh1.reshape(1, 1, 64),
                            998, True)
    sc2, sh2 = _bn_stats(s_2, q_2, bsz * 996.0, pg2, pbe2)
    y3, s_3, q_3 = _conv_bn(y2, w3m, pb3.reshape(1, 1, 128),
                            sc2.reshape(1, 1, 128), sh2.reshape(1, 1, 128),
                            996, True)
    sc3, sh3 = _bn_stats(s_3, q_3, bsz * 994.0, pg3, pbe3)
    prot_vec = _bn_relu_max(y3, sc3.reshape(1, 1, 128),
                            sh3.reshape(1, 1, 128), 994)

    oWp = jnp.concatenate([oW, jnp.zeros((64, 7), jnp.float32)], axis=1)
    out = _fusion_head(summed, inv_counts, row(g3), row(be3), prot_vec,
                       fW1[:128], fW1[128:], row(fb1), row(fg1), row(fbe1),
                       fW2, row(fb2), row(fg2), row(fbe2), oWp)
    return out[:, :1] + ob[None, :]
